# baseline (speedup 1.0000x reference)
import functools

import jax
import jax.numpy as jnp
from jax import lax
from jax.experimental import pallas as pl
from jax.experimental.pallas import tpu as pltpu
from jax.experimental.pallas import tpu_sc as plsc

N_USERS = 1000000
N_ITEMS = 1000000
EMB_DIM = 32
N_FACTORS = 4
DECAY = 1e-3
COR_WEIGHT = 0.01
BATCH_SIZE = 16384
COR_BATCH = 1024

PACK = 128 // EMB_DIM
NROW = N_USERS // PACK

NC, NS = 2, 16
NW = NC * NS
CHUNK = 128
BPW = BATCH_SIZE // NW
CPW = COR_BATCH // NW

_F32 = jnp.float32


def _sc_gather_body(users_h, pos_h, neg_h, coru_h, cori_h,
                    uall_h, iall_h, uego_h, iego_h,
                    ue_o, pe_o, ne_o, uo_o, po_o, no_o, cor_o,
                    idx_v, rows_v, idxs_v, rowss_v, sem):
    wid = lax.axis_index("s") * NC + lax.axis_index("c")

    def run(idx_h, table_outs, bpw, iv, rv, ch, out_base):
        base = wid * bpw
        for c in range(bpw // ch):
            off = base + c * ch
            pltpu.sync_copy(idx_h.at[pl.ds(off, ch)], iv)
            for tab_h, out_h in table_outs:
                pltpu.async_copy(tab_h.at[iv], rv, sem).wait()
                pltpu.sync_copy(rv, out_h.at[pl.ds(out_base + off, ch)])

    run(users_h, [(uall_h, ue_o), (uego_h, uo_o)], BPW, idx_v, rows_v, CHUNK, 0)
    run(pos_h, [(iall_h, pe_o), (iego_h, po_o)], BPW, idx_v, rows_v, CHUNK, 0)
    run(neg_h, [(iall_h, ne_o), (iego_h, no_o)], BPW, idx_v, rows_v, CHUNK, 0)
    run(coru_h, [(uall_h, cor_o)], CPW, idxs_v, rowss_v, CPW, 0)
    run(cori_h, [(iall_h, cor_o)], CPW, idxs_v, rowss_v, CPW, COR_BATCH)


_big = jax.ShapeDtypeStruct((BATCH_SIZE, 128), _F32)
_cor = jax.ShapeDtypeStruct((2 * COR_BATCH, 128), _F32)


@functools.cache
def _sc_gather():
    return pl.kernel(
        _sc_gather_body,
        out_type=(_big, _big, _big, _big, _big, _big, _cor),
        mesh=plsc.VectorSubcoreMesh(core_axis_name="c", subcore_axis_name="s",
                                    num_cores=NC, num_subcores=NS),
        scratch_types=[
            pltpu.VMEM((CHUNK,), jnp.int32),
            pltpu.VMEM((CHUNK, 128), _F32),
            pltpu.VMEM((CPW,), jnp.int32),
            pltpu.VMEM((CPW, 128), _F32),
            pltpu.SemaphoreType.DMA,
        ],
    )


N2 = 2 * COR_BATCH
GRID = 8
CB = N2 // GRID
BB = BATCH_SIZE // GRID
FD = EMB_DIM // N_FACTORS

_A_MF = 0
_A_REG = 1
_A_S = 2
_A_SELF = 6
_A_Q = 10
_A_CROSS = 14
_A_P = 17
_N_ACC = 20


def _select32(blk128, qcol):
    out = None
    for q in range(PACK):
        m = jnp.where(qcol == float(q), 1.0, 0.0)
        part = m * blk128[:, q * EMB_DIM:(q + 1) * EMB_DIM]
        out = part if out is None else out + part
    return out


def _tc_body(ue, pe, ne, uo, po, no, uq, pq, nq, ui128, corq,
             out_ref, acc, ui_s):
    i = pl.program_id(0)

    @pl.when(i == 0)
    def _init():
        for j in range(_N_ACC):
            acc[j] = 0.0
        ui_s[...] = _select32(ui128[...], corq[...])

    u = _select32(ue[...], uq[...])
    p = _select32(pe[...], pq[...])
    nn = _select32(ne[...], nq[...])
    pos_s = jnp.sum(u * p, axis=1)
    neg_s = jnp.sum(u * nn, axis=1)
    d = neg_s - pos_s
    sp = jnp.maximum(d, 0.0) + jnp.log(1.0 + jnp.exp(-jnp.abs(d)))
    acc[_A_MF] = acc[_A_MF] + jnp.sum(sp)
    uo32 = _select32(uo[...], uq[...])
    po32 = _select32(po[...], pq[...])
    no32 = _select32(no[...], nq[...])
    reg = jnp.sum(uo32 * uo32) + jnp.sum(po32 * po32) + jnp.sum(no32 * no32)
    acc[_A_REG] = acc[_A_REG] + reg

    xb = ui_s[pl.ds(i * CB, CB), :]
    xf = ui_s[...]
    ds = []
    rss = []
    for k in range(N_FACTORS):
        xk = xb[:, k * FD:(k + 1) * FD]
        fk = xf[:, k * FD:(k + 1) * FD]
        r_full = jnp.sum(fk * fk, axis=1)
        r_blk = jnp.sum(xk * xk, axis=1)
        g = lax.dot_general(xk, fk, (((1,), (1,)), ((), ())),
                            preferred_element_type=_F32)
        d2 = r_blk[:, None] - 2.0 * g + r_full[None, :]
        dmat = jnp.sqrt(jnp.maximum(d2, 0.0) + 1e-8)
        rs = jnp.sum(dmat, axis=1)
        acc[_A_S + k] = acc[_A_S + k] + jnp.sum(rs)
        acc[_A_SELF + k] = acc[_A_SELF + k] + jnp.sum(rs * rs)
        acc[_A_Q + k] = acc[_A_Q + k] + jnp.sum(dmat * dmat)
        ds.append(dmat)
        rss.append(rs)
    for pp in range(N_FACTORS - 1):
        acc[_A_CROSS + pp] = acc[_A_CROSS + pp] + jnp.sum(rss[pp] * rss[pp + 1])
        acc[_A_P + pp] = acc[_A_P + pp] + jnp.sum(ds[pp] * ds[pp + 1])

    @pl.when(i == GRID - 1)
    def _fin():
        n = float(N2)
        mf = acc[_A_MF] / float(BATCH_SIZE)
        emb = DECAY * (acc[_A_REG] / 2.0) / float(BATCH_SIZE)

        def centered_sum(prod, cross, sa, sb):
            return prod - (2.0 / n) * cross + sa * sb / (n * n)

        def dcov(csum):
            return jnp.sqrt(jnp.maximum(csum / (n * n), 0.0) + 1e-8)

        cor = jnp.float32(0.0)
        for pp in range(N_FACTORS - 1):
            a, b = pp, pp + 1
            s12 = centered_sum(acc[_A_P + pp], acc[_A_CROSS + pp],
                               acc[_A_S + a], acc[_A_S + b])
            s11 = centered_sum(acc[_A_Q + a], acc[_A_SELF + a],
                               acc[_A_S + a], acc[_A_S + a])
            s22 = centered_sum(acc[_A_Q + b], acc[_A_SELF + b],
                               acc[_A_S + b], acc[_A_S + b])
            d12, d11, d22 = dcov(s12), dcov(s11), dcov(s22)
            cor = cor + d12 / (jnp.sqrt(jnp.maximum(d11 * d22, 0.0)) + 1e-10)
        cor_loss = COR_WEIGHT * cor / ((N_FACTORS + 1.0) * N_FACTORS / 2.0)
        out_ref[0] = mf
        out_ref[1] = emb
        out_ref[2] = cor_loss
        out_ref[3] = mf + emb + cor_loss


_big_spec = pl.BlockSpec((BB, 128), lambda i: (i, 0))
_q_spec = pl.BlockSpec((BB, 1), lambda i: (i, 0))

_tc_losses = pl.pallas_call(
    _tc_body,
    grid=(GRID,),
    in_specs=[
        _big_spec, _big_spec, _big_spec, _big_spec, _big_spec, _big_spec,
        _q_spec, _q_spec, _q_spec,
        pl.BlockSpec((N2, 128), lambda i: (0, 0)),
        pl.BlockSpec((N2, 1), lambda i: (0, 0)),
    ],
    out_specs=pl.BlockSpec(memory_space=pltpu.SMEM),
    out_shape=jax.ShapeDtypeStruct((4,), _F32),
    scratch_shapes=[pltpu.SMEM((_N_ACC,), _F32),
                    pltpu.VMEM((N2, EMB_DIM), _F32)],
)


def kernel(users, pos_items, neg_items, cor_users, cor_items,
           user_embedding, item_embedding,
           user_all_embeddings, item_all_embeddings):
    users = users.astype(jnp.int32)
    pos_items = pos_items.astype(jnp.int32)
    neg_items = neg_items.astype(jnp.int32)
    cor_users = cor_users.astype(jnp.int32)
    cor_items = cor_items.astype(jnp.int32)

    uall4 = user_all_embeddings.reshape(NROW, 128)
    iall4 = item_all_embeddings.reshape(NROW, 128)
    uego4 = user_embedding.reshape(NROW, 128)
    iego4 = item_embedding.reshape(NROW, 128)

    ue, pe, ne, uo, po, no, cor128 = _sc_gather()(
        users // PACK, pos_items // PACK, neg_items // PACK,
        cor_users // PACK, cor_items // PACK,
        uall4, iall4, uego4, iego4)

    uq = (users % PACK).astype(_F32).reshape(BATCH_SIZE, 1)
    pq = (pos_items % PACK).astype(_F32).reshape(BATCH_SIZE, 1)
    nq = (neg_items % PACK).astype(_F32).reshape(BATCH_SIZE, 1)
    corq = jnp.concatenate([cor_users % PACK, cor_items % PACK]) \
        .astype(_F32).reshape(N2, 1)

    out = _tc_losses(ue, pe, ne, uo, po, no, uq, pq, nq, cor128, corq)
    mf_loss, emb_loss, cor_loss, loss = out[0], out[1], out[2], out[3]
    return (mf_loss, emb_loss, cor_loss, loss)


# speedup vs baseline: 9.2141x; 9.2141x over previous
import jax
import jax.numpy as jnp
from jax import lax
from jax.experimental import pallas as pl
from jax.experimental.pallas import tpu as pltpu

N_USERS = 1000000
N_ITEMS = 1000000
EMB_DIM = 32
N_FACTORS = 4
DECAY = 1e-3
COR_WEIGHT = 0.01
BATCH_SIZE = 16384
COR_BATCH = 1024

_F32 = jnp.float32

N2 = 2 * COR_BATCH
GRID = 8
CB = N2 // GRID
FD = EMB_DIM // N_FACTORS

_A_S = 0
_A_SELF = 4
_A_Q = 8
_A_CROSS = 12
_A_P = 15
_N_ACC = 18


def _tc_cor_body(uib, uiT, out_ref, acc):
    i = pl.program_id(0)

    @pl.when(i == 0)
    def _init():
        for j in range(_N_ACC):
            acc[j] = 0.0

    xb = uib[...]
    xt = uiT[...]
    ds = []
    rss = []
    for k in range(N_FACTORS):
        xk = xb[:, k * FD:(k + 1) * FD]
        tk = xt[k * FD:(k + 1) * FD, :]
        r_full = jnp.sum(tk * tk, axis=0)
        r_blk = jnp.sum(xk * xk, axis=1)
        g = jnp.dot(xk, tk, preferred_element_type=_F32)
        d2 = r_blk[:, None] - 2.0 * g + r_full[None, :]
        dmat = jnp.sqrt(jnp.maximum(d2, 0.0) + 1e-8)
        rs = jnp.sum(dmat, axis=1)
        acc[_A_S + k] = acc[_A_S + k] + jnp.sum(rs)
        acc[_A_SELF + k] = acc[_A_SELF + k] + jnp.sum(rs * rs)
        acc[_A_Q + k] = acc[_A_Q + k] + jnp.sum(dmat * dmat)
        ds.append(dmat)
        rss.append(rs)
    for pp in range(N_FACTORS - 1):
        acc[_A_CROSS + pp] = acc[_A_CROSS + pp] + jnp.sum(rss[pp] * rss[pp + 1])
        acc[_A_P + pp] = acc[_A_P + pp] + jnp.sum(ds[pp] * ds[pp + 1])

    @pl.when(i == GRID - 1)
    def _fin():
        n = float(N2)

        def centered_sum(prod, cross, sa, sb):
            return prod - (2.0 / n) * cross + sa * sb / (n * n)

        def dcov(csum):
            return jnp.sqrt(jnp.maximum(csum / (n * n), 0.0) + 1e-8)

        cor = jnp.float32(0.0)
        for pp in range(N_FACTORS - 1):
            a, b = pp, pp + 1
            s12 = centered_sum(acc[_A_P + pp], acc[_A_CROSS + pp],
                               acc[_A_S + a], acc[_A_S + b])
            s11 = centered_sum(acc[_A_Q + a], acc[_A_SELF + a],
                               acc[_A_S + a], acc[_A_S + a])
            s22 = centered_sum(acc[_A_Q + b], acc[_A_SELF + b],
                               acc[_A_S + b], acc[_A_S + b])
            d12, d11, d22 = dcov(s12), dcov(s11), dcov(s22)
            cor = cor + d12 / (jnp.sqrt(jnp.maximum(d11 * d22, 0.0)) + 1e-10)
        out_ref[0] = COR_WEIGHT * cor / ((N_FACTORS + 1.0) * N_FACTORS / 2.0)


_tc_cor = pl.pallas_call(
    _tc_cor_body,
    grid=(GRID,),
    in_specs=[
        pl.BlockSpec((CB, EMB_DIM), lambda i: (i, 0)),
        pl.BlockSpec((EMB_DIM, N2), lambda i: (0, 0)),
    ],
    out_specs=pl.BlockSpec(memory_space=pltpu.SMEM),
    out_shape=jax.ShapeDtypeStruct((1,), _F32),
    scratch_shapes=[pltpu.SMEM((_N_ACC,), _F32)],
)


def _tc_bpr_body(ueT, peT, neT, uoT, poT, noT, out_ref):
    u = ueT[...]
    pos_s = jnp.sum(u * peT[...], axis=0, keepdims=True)
    neg_s = jnp.sum(u * neT[...], axis=0, keepdims=True)
    d = neg_s - pos_s
    sp = jnp.maximum(d, 0.0) + jnp.log(1.0 + jnp.exp(-jnp.abs(d)))
    uo = uoT[...]
    po = poT[...]
    no = noT[...]
    reg = jnp.sum(uo * uo) + jnp.sum(po * po) + jnp.sum(no * no)
    out_ref[0] = jnp.sum(sp) / float(BATCH_SIZE)
    out_ref[1] = DECAY * (reg / 2.0) / float(BATCH_SIZE)


_tc_bpr = pl.pallas_call(
    _tc_bpr_body,
    out_specs=pl.BlockSpec(memory_space=pltpu.SMEM),
    out_shape=jax.ShapeDtypeStruct((2,), _F32),
)


def kernel(users, pos_items, neg_items, cor_users, cor_items,
           user_embedding, item_embedding,
           user_all_embeddings, item_all_embeddings):
    cu = jnp.take(user_all_embeddings, cor_users, axis=0)
    ci = jnp.take(item_all_embeddings, cor_items, axis=0)
    ue = jnp.take(user_all_embeddings, users, axis=0)
    pe = jnp.take(item_all_embeddings, pos_items, axis=0)
    ne = jnp.take(item_all_embeddings, neg_items, axis=0)
    uo = jnp.take(user_embedding, users, axis=0)
    po = jnp.take(item_embedding, pos_items, axis=0)
    no = jnp.take(item_embedding, neg_items, axis=0)

    ui = jnp.concatenate([cu, ci], axis=0)
    uiT = jnp.concatenate([cu.T, ci.T], axis=1)

    cor_loss = _tc_cor(ui, uiT)[0]
    mfemb = _tc_bpr(ue.T, pe.T, ne.T, uo.T, po.T, no.T)
    mf_loss, emb_loss = mfemb[0], mfemb[1]
    return (mf_loss, emb_loss, cor_loss, mf_loss + emb_loss + cor_loss)


# speedup vs baseline: 10.2331x; 1.1106x over previous
import jax
import jax.numpy as jnp
from jax import lax
from jax.experimental import pallas as pl
from jax.experimental.pallas import tpu as pltpu

N_USERS = 1000000
N_ITEMS = 1000000
EMB_DIM = 32
N_FACTORS = 4
DECAY = 1e-3
COR_WEIGHT = 0.01
BATCH_SIZE = 16384
COR_BATCH = 1024

_F32 = jnp.float32

N2 = 2 * COR_BATCH
GRID = 8
CB = N2 // GRID
FD = EMB_DIM // N_FACTORS

_A_S = 0
_A_SELF = 4
_A_Q = 8
_A_CROSS = 12
_A_P = 15
_N_ACC = 18


def _tc_cor_body(uib, uiT, out_ref, acc):
    i = pl.program_id(0)

    @pl.when(i == 0)
    def _init():
        for j in range(_N_ACC):
            acc[j] = 0.0

    xb = uib[...]
    xt = uiT[...]
    ds = []
    rss = []
    for k in range(N_FACTORS):
        xk = xb[:, k * FD:(k + 1) * FD]
        tk = xt[k * FD:(k + 1) * FD, :]
        r_full = jnp.sum(tk * tk, axis=0)
        r_blk = jnp.sum(xk * xk, axis=1)
        g = jnp.dot(xk, tk, preferred_element_type=_F32)
        d2 = r_blk[:, None] - 2.0 * g + r_full[None, :]
        dmat = jnp.sqrt(jnp.maximum(d2, 0.0) + 1e-8)
        rs = jnp.sum(dmat, axis=1)
        acc[_A_S + k] = acc[_A_S + k] + jnp.sum(rs)
        acc[_A_SELF + k] = acc[_A_SELF + k] + jnp.sum(rs * rs)
        acc[_A_Q + k] = acc[_A_Q + k] + jnp.sum(dmat * dmat)
        ds.append(dmat)
        rss.append(rs)
    for pp in range(N_FACTORS - 1):
        acc[_A_CROSS + pp] = acc[_A_CROSS + pp] + jnp.sum(rss[pp] * rss[pp + 1])
        acc[_A_P + pp] = acc[_A_P + pp] + jnp.sum(ds[pp] * ds[pp + 1])

    @pl.when(i == GRID - 1)
    def _fin():
        n = float(N2)

        def centered_sum(prod, cross, sa, sb):
            return prod - (2.0 / n) * cross + sa * sb / (n * n)

        def dcov(csum):
            return jnp.sqrt(jnp.maximum(csum / (n * n), 0.0) + 1e-8)

        cor = jnp.float32(0.0)
        for pp in range(N_FACTORS - 1):
            a, b = pp, pp + 1
            s12 = centered_sum(acc[_A_P + pp], acc[_A_CROSS + pp],
                               acc[_A_S + a], acc[_A_S + b])
            s11 = centered_sum(acc[_A_Q + a], acc[_A_SELF + a],
                               acc[_A_S + a], acc[_A_S + a])
            s22 = centered_sum(acc[_A_Q + b], acc[_A_SELF + b],
                               acc[_A_S + b], acc[_A_S + b])
            d12, d11, d22 = dcov(s12), dcov(s11), dcov(s22)
            cor = cor + d12 / (jnp.sqrt(jnp.maximum(d11 * d22, 0.0)) + 1e-10)
        out_ref[0] = COR_WEIGHT * cor / ((N_FACTORS + 1.0) * N_FACTORS / 2.0)


_tc_cor = pl.pallas_call(
    _tc_cor_body,
    grid=(GRID,),
    in_specs=[
        pl.BlockSpec((CB, EMB_DIM), lambda i: (i, 0)),
        pl.BlockSpec((EMB_DIM, N2), lambda i: (0, 0)),
    ],
    out_specs=pl.BlockSpec(memory_space=pltpu.SMEM),
    out_shape=jax.ShapeDtypeStruct((1,), _F32),
    scratch_shapes=[pltpu.SMEM((_N_ACC,), _F32)],
)


def _tc_bpr_body(ueT, peT, neT, uoT, poT, noT, out_ref):
    u = ueT[...]
    pos_s = jnp.sum(u * peT[...], axis=0, keepdims=True)
    neg_s = jnp.sum(u * neT[...], axis=0, keepdims=True)
    d = neg_s - pos_s
    sp = jnp.maximum(d, 0.0) + jnp.log(1.0 + jnp.exp(-jnp.abs(d)))
    uo = uoT[...]
    po = poT[...]
    no = noT[...]
    reg = jnp.sum(uo * uo) + jnp.sum(po * po) + jnp.sum(no * no)
    out_ref[0] = jnp.sum(sp) / float(BATCH_SIZE)
    out_ref[1] = DECAY * (reg / 2.0) / float(BATCH_SIZE)


_tc_bpr = pl.pallas_call(
    _tc_bpr_body,
    out_specs=pl.BlockSpec(memory_space=pltpu.SMEM),
    out_shape=jax.ShapeDtypeStruct((2,), _F32),
)


def kernel(users, pos_items, neg_items, cor_users, cor_items,
           user_embedding, item_embedding,
           user_all_embeddings, item_all_embeddings):
    cu = user_all_embeddings.at[cor_users].get(mode='promise_in_bounds')
    ci = item_all_embeddings.at[cor_items].get(mode='promise_in_bounds')
    ue = user_all_embeddings.at[users].get(mode='promise_in_bounds')
    pe = item_all_embeddings.at[pos_items].get(mode='promise_in_bounds')
    ne = item_all_embeddings.at[neg_items].get(mode='promise_in_bounds')
    uo = user_embedding.at[users].get(mode='promise_in_bounds')
    po = item_embedding.at[pos_items].get(mode='promise_in_bounds')
    no = item_embedding.at[neg_items].get(mode='promise_in_bounds')

    ui = jnp.concatenate([cu, ci], axis=0)
    uiT = jnp.concatenate([cu.T, ci.T], axis=1)

    cor_loss = _tc_cor(ui, uiT)[0]
    mfemb = _tc_bpr(ue.T, pe.T, ne.T, uo.T, po.T, no.T)
    mf_loss, emb_loss = mfemb[0], mfemb[1]
    return (mf_loss, emb_loss, cor_loss, mf_loss + emb_loss + cor_loss)
